# baseline (speedup 1.0000x reference)
import jax
import jax.numpy as jnp
from jax.experimental import pallas as pl
from jax.experimental.pallas import tpu as pltpu


def _sage_block_kernel(adj_ref, h_ref, wt_ref, b_ref, out_ref):
    y = jnp.dot(adj_ref[...], h_ref[...], preferred_element_type=jnp.float32)
    out_ref[...] = (
        jnp.dot(y, wt_ref[...], preferred_element_type=jnp.float32) + b_ref[...]
    )


def kernel(adj, h, W, bias):
    n, d_in = h.shape
    d_out = W.shape[0]

    block_rows = n
    for cand in range(min(512, n), 7, -1):
        if n % cand == 0 and cand % 8 == 0:
            block_rows = cand
            break

    wt = W.T
    bias2d = bias.reshape(1, d_out)

    out = pl.pallas_call(
        _sage_block_kernel,
        grid=(n // block_rows,),
        in_specs=[
            pl.BlockSpec((block_rows, n), lambda i: (i, 0)),
            pl.BlockSpec((n, d_in), lambda i: (0, 0)),
            pl.BlockSpec((d_in, d_out), lambda i: (0, 0)),
            pl.BlockSpec((1, d_out), lambda i: (0, 0)),
        ],
        out_specs=pl.BlockSpec((block_rows, d_out), lambda i: (i, 0)),
        out_shape=jax.ShapeDtypeStruct((n, d_out), jnp.float32),
        compiler_params=pltpu.CompilerParams(
            dimension_semantics=("arbitrary",),
        ),
    )(adj, h, wt, bias2d)
    return out
